# baseline (speedup 1.0000x reference)
import functools

import jax
import jax.numpy as jnp
from jax import lax
from jax.experimental import pallas as pl
from jax.experimental.pallas import tpu as pltpu

MAX_LEN = 8192
D_MODEL = 1024
ROWS_PER_STEP = 512


def _tc_body(idx_ref, table_ref, out_ref):
    def row(r, carry):
        out_ref[r] = table_ref[idx_ref[r]]
        return carry
    lax.fori_loop(0, ROWS_PER_STEP, row, 0, unroll=8)


@functools.cache
def _make_tc_lookup(B):
    grid = B // ROWS_PER_STEP
    return pl.pallas_call(
        _tc_body,
        grid=(grid,),
        in_specs=[
            pl.BlockSpec((ROWS_PER_STEP,), lambda i: (i,),
                         memory_space=pltpu.SMEM),
            pl.BlockSpec((MAX_LEN, 8, 128), lambda i: (0, 0, 0)),
        ],
        out_specs=pl.BlockSpec((ROWS_PER_STEP, 8, 128), lambda i: (i, 0, 0)),
        out_shape=jax.ShapeDtypeStruct((B, 8, 128), jnp.float32),
    )


def kernel(x, table):
    B = x.size
    idx = jnp.reshape(x.astype(jnp.int32), (B,))
    out = _make_tc_lookup(B)(idx, jnp.reshape(table, (MAX_LEN, 8, 128)))
    return jnp.reshape(out, x.shape + (D_MODEL,))
